# baseline (speedup 1.0000x reference)
import functools

import jax
import jax.numpy as jnp
from jax.experimental import pallas as pl
from jax.experimental.pallas import tpu as pltpu

B = 4096
D = 128
X = 100000
XPAD = 100352
BT = 256
XT = 2048
NT = XPAD // XT
G1 = 16
NB1 = XPAD // G1
NB2 = 7 * 128
G2 = 7
K = 100
NEG = -jnp.inf
IBIG = 2**30


def _matmul_block(q_ref, e_ref, s_ref, m1_ref, m2_ref):
    u = pl.program_id(1)
    a = pl.program_id(2)
    t = a * 7 + u
    s = jnp.dot(q_ref[...], e_ref[...], preferred_element_type=jnp.float32)
    col = t * XT + jax.lax.broadcasted_iota(jnp.int32, (BT, XT), 1)
    s = jnp.where(col < X, s, NEG)
    s_ref[...] = s
    m1 = jnp.max(s.reshape(BT, G1, 128), axis=1)
    m1_ref[...] = m1

    @pl.when(a == 0)
    def _init():
        m2_ref[...] = m1

    @pl.when(a > 0)
    def _acc():
        m2_ref[...] = jnp.maximum(m2_ref[...], m1)


def _pop_loop(v, pay, pops):
    lane = jax.lax.broadcasted_iota(jnp.int32, (BT, 128), 1)

    def body(p, carry):
        v, accv, acci = carry
        m = jnp.max(v, axis=1, keepdims=True)
        sel = v == m
        pid = jnp.min(jnp.where(sel, pay, IBIG), axis=1, keepdims=True)
        v = jnp.where(pay == pid, NEG, v)
        accv = jnp.where(lane == p, m, accv)
        acci = jnp.where(lane == p, pid, acci)
        return v, accv, acci

    accv0 = jnp.full((BT, 128), NEG, jnp.float32)
    acci0 = jnp.full((BT, 128), -1, jnp.int32)
    return jax.lax.fori_loop(0, pops, body, (v, accv0, acci0))[1:]


def _popk_block(v_ref, p_ref, vout_ref, iout_ref, *, pops):
    accv, acci = _pop_loop(v_ref[...], p_ref[...], pops)
    vout_ref[...] = accv
    iout_ref[...] = acci


def _popk(vals, payload, width, pops):
    return pl.pallas_call(
        functools.partial(_popk_block, pops=pops),
        grid=(B // BT,),
        in_specs=[
            pl.BlockSpec((BT, width), lambda i: (i, 0)),
            pl.BlockSpec((BT, width), lambda i: (i, 0)),
        ],
        out_specs=[
            pl.BlockSpec((BT, 128), lambda i: (i, 0)),
            pl.BlockSpec((BT, 128), lambda i: (i, 0)),
        ],
        out_shape=[
            jax.ShapeDtypeStruct((B, 128), jnp.float32),
            jax.ShapeDtypeStruct((B, 128), jnp.int32),
        ],
    )(vals, payload)


def _pop2_block(m2_ref, vout_ref, iout_ref, *, pops):
    m2 = jnp.concatenate(
        [m2_ref[...], jnp.full((BT, 1024 - NB2), NEG, jnp.float32)], axis=1)
    pay = jax.lax.broadcasted_iota(jnp.int32, (BT, 1024), 1)
    accv, acci = _pop_loop(m2, pay, pops)
    vout_ref[...] = accv
    iout_ref[...] = acci


def kernel(query_embeddings, item_embeddings_t, ids, k):
    e_pad = jnp.pad(item_embeddings_t, ((0, 0), (0, XPAD - X)))
    scores, m1, m2 = pl.pallas_call(
        _matmul_block,
        grid=(B // BT, 7, 7),
        in_specs=[
            pl.BlockSpec((BT, D), lambda i, u, a: (i, 0)),
            pl.BlockSpec((D, XT), lambda i, u, a: (0, a * 7 + u)),
        ],
        out_specs=[
            pl.BlockSpec((BT, XT), lambda i, u, a: (i, a * 7 + u)),
            pl.BlockSpec((BT, 128), lambda i, u, a: (i, a * 7 + u)),
            pl.BlockSpec((BT, 128), lambda i, u, a: (i, u)),
        ],
        out_shape=[
            jax.ShapeDtypeStruct((B, XPAD), jnp.float32),
            jax.ShapeDtypeStruct((B, NB1), jnp.float32),
            jax.ShapeDtypeStruct((B, NB2), jnp.float32),
        ],
        compiler_params=pltpu.CompilerParams(
            dimension_semantics=("parallel", "arbitrary", "arbitrary"),
        ),
    )(query_embeddings, e_pad)

    _, bids2 = pl.pallas_call(
        functools.partial(_pop2_block, pops=K),
        grid=(B // BT,),
        in_specs=[pl.BlockSpec((BT, NB2), lambda i: (i, 0))],
        out_specs=[
            pl.BlockSpec((BT, 128), lambda i: (i, 0)),
            pl.BlockSpec((BT, 128), lambda i: (i, 0)),
        ],
        out_shape=[
            jax.ShapeDtypeStruct((B, 128), jnp.float32),
            jax.ShapeDtypeStruct((B, 128), jnp.int32),
        ],
    )(m2)
    bids2 = bids2[:, :K]

    i1 = (bids2[:, :, None]
          + (NB2 * jnp.arange(G2, dtype=jnp.int32))[None, None, :]).reshape(B, K * G2)
    c1 = jnp.take_along_axis(m1, i1, axis=1)
    c1 = jnp.concatenate(
        [c1, jnp.full((B, 1024 - K * G2), NEG, jnp.float32)], axis=1)
    i1 = jnp.concatenate(
        [i1, jnp.full((B, 1024 - K * G2), IBIG, jnp.int32)], axis=1)

    _, bids1 = _popk(c1, i1, 1024, K)
    bids1 = bids1[:, :K]

    base = (bids1 // 128) * XT + (bids1 % 128)
    i0 = (base[:, :, None]
          + (128 * jnp.arange(G1, dtype=jnp.int32))[None, None, :]).reshape(B, K * G1)
    c0 = jnp.take_along_axis(scores, i0, axis=1)
    c0 = jnp.concatenate(
        [c0, jnp.full((B, 1664 - K * G1), NEG, jnp.float32)], axis=1)
    i0 = jnp.concatenate(
        [i0, jnp.full((B, 1664 - K * G1), IBIG, jnp.int32)], axis=1)

    svals, scols = _popk(c0, i0, 1664, K)
    top_scores = svals[:, :K]
    top_cols = scols[:, :K]
    top_ids = jnp.take_along_axis(
        ids, top_cols.reshape(1, B * K), axis=1).reshape(B, K)
    return top_ids, top_scores
